# baseline (speedup 1.0000x reference)
import functools

import jax
import jax.numpy as jnp
from jax import lax
from jax.experimental import pallas as pl
from jax.experimental.pallas import tpu as pltpu

SEQ = 4096
NB = 64
NC = 128
UNROLL = 64


def _flip_rows(x):
    u = x.shape[0]
    return jnp.concatenate([x[i:i + 1] for i in range(u - 1, -1, -1)], axis=0)


def _scan_kernel(x2_ref, w0_ref, b0_ref, whh0_ref, wa_ref, wb_ref, b1_ref,
                 whh1_ref, o1_ref, o0_scr, zb0, zb1, *, seq, unroll):
    u = unroll
    ng = seq // u
    w0 = w0_ref[...]
    b0 = b0_ref[...]
    wa = wa_ref[...]
    wb = wb_ref[...]
    b1 = b1_ref[...]
    w4_0 = tuple(whh0_ref[g] for g in range(4))
    w4_1 = tuple(whh1_ref[g] for g in range(4))


    def build0(g, buf):
        gc = jnp.minimum(g, ng - 1)
        base = gc * u
        xg = x2_ref[pl.ds(base, u), :, :]
        zg = jnp.broadcast_to(b0[None], (u, 4, NC))
        for d in range(3):
            zg = zg + xg[:, d, None, :] * w0[None, :, d, :]
        buf[...] = zg.reshape(u, 4, 1, NC)

    def build1(g, buf):
        gc = jnp.minimum(g, ng - 1)
        base = gc * u
        rbase = seq - u - base
        a_blk = o0_scr[pl.ds(base, u), :]
        r_blk = _flip_rows(o0_scr[pl.ds(rbase, u), :])
        rsh = jnp.concatenate([r_blk[:, 64:], r_blk[:, :64]], axis=-1)
        zg = (jnp.broadcast_to(b1[None], (u, 4, NC))
              + a_blk[:, None, :] * wa[None]
              + rsh[:, None, :] * wb[None])
        buf[...] = zg.reshape(u, 4, 1, NC)

    def steps(buf, st, w4, out_scale=None):
        tc, top1, c = st
        w_i, w_f, w_g, w_o = w4
        hs = []
        blocks = []
        for j in range(u):
            q_i = top1 * w_i
            q_f = top1 * w_f
            q_g = top1 * w_g
            q_o = top1 * w_o
            ch = c * 0.5
            tg = jnp.tanh(buf[j, 2] + tc * q_g)
            ti = jnp.tanh(buf[j, 0] + tc * q_i)
            tf = jnp.tanh(buf[j, 1] + tc * q_f)
            to = jnp.tanh(buf[j, 3] + tc * q_o)
            c = ch * (tf + 1.0) + (tg * 0.5) * (ti + 1.0)
            top1 = to + 1.0
            tc = jnp.tanh(c)
            h2 = tc * top1
            hs.append(h2 if out_scale is None else h2 * out_scale)
            if len(hs) == 8:
                blocks.append(jnp.concatenate(hs, axis=0))
                hs = []
        return blocks, (tc, top1, c)

    zv = jnp.zeros((1, NC), jnp.float32)
    ov = jnp.ones((1, NC), jnp.float32)

    def l0_half(g, buf_run, buf_next, st):
        build0(g + 1, buf_next)
        blocks, st = steps(buf_run, st, w4_0)
        base = g * u
        for k, blk in enumerate(blocks):
            o0_scr[pl.ds(base + 8 * k, 8), :] = blk
        return st

    def l0_body(gg, st):
        g = gg * 2
        st = l0_half(g, zb0, zb1, st)
        st = l0_half(g + 1, zb1, zb0, st)
        return st

    build0(0, zb0)
    lax.fori_loop(0, ng // 2, l0_body, (zv, ov, zv))

    def l1_half(g, buf_run, buf_next, st):
        build1(g + 1, buf_next)
        blocks, st = steps(buf_run, st, w4_1, out_scale=0.5)
        base = g * u
        rbase = seq - u - base
        for k, blk in enumerate(blocks):
            o1_ref[pl.ds(base + 8 * k, 8), 0:64] = blk[:, 0:64]
            o1_ref[pl.ds(rbase + u - 8 - 8 * k, 8), 64:128] = (
                _flip_rows(blk[:, 64:128]))
        return st

    def l1_body(gg, st):
        g = gg * 2
        st = l1_half(g, zb0, zb1, st)
        st = l1_half(g + 1, zb1, zb0, st)
        return st

    build1(0, zb0)
    lax.fori_loop(0, ng // 2, l1_body, (zv, ov, zv))


def _mlp_kernel(x_ref, w1_ref, b1_ref, w2_ref, b2_ref, w3_ref, b3_ref,
                o_ref, acc_ref, *, kt):
    k = pl.program_id(0)

    @pl.when(k == 0)
    def _():
        acc_ref[...] = jnp.zeros_like(acc_ref)

    acc_ref[...] += jnp.dot(x_ref[...], w1_ref[...],
                            preferred_element_type=jnp.float32)

    @pl.when(k == kt - 1)
    def _():
        h1 = acc_ref[...] + b1_ref[...]
        h2 = jnp.maximum(
            jnp.dot(h1, w2_ref[...], preferred_element_type=jnp.float32)
            + b2_ref[...], 0.0)
        y = jnp.dot(h2, w3_ref[...],
                    preferred_element_type=jnp.float32) + b3_ref[...]
        o_ref[...] = y


def _halves(f, b, shape):
    return jnp.concatenate([jnp.broadcast_to(f, shape),
                            jnp.broadcast_to(b, shape)], axis=-1)


def kernel(pos, batch,
           lstm_0_f_w_ih, lstm_0_f_w_hh, lstm_0_f_b_ih, lstm_0_f_b_hh,
           lstm_0_b_w_ih, lstm_0_b_w_hh, lstm_0_b_b_ih, lstm_0_b_b_hh,
           lstm_1_f_w_ih, lstm_1_f_w_hh, lstm_1_f_b_ih, lstm_1_f_b_hh,
           lstm_1_b_w_ih, lstm_1_b_w_hh, lstm_1_b_b_ih, lstm_1_b_b_hh,
           lin1_w, lin1_b, lin2_w, lin2_b, lin3_w, lin3_b):
    seq = SEQ
    x = pos.reshape(NB, seq, 3).astype(jnp.float32)
    xt = jnp.transpose(x, (1, 2, 0))
    x2 = jnp.concatenate([xt, xt[::-1]], axis=-1)

    gs = jnp.array([0.5, 0.5, 1.0, 0.5], jnp.float32)

    def dparams(w_ih, w_hh, b_ih, b_hh):
        return (w_ih * gs[:, None],
                (b_ih + b_hh) * gs,
                w_hh[:, 0] * gs)

    w0f, b0f, wh0f = dparams(lstm_0_f_w_ih, lstm_0_f_w_hh,
                             lstm_0_f_b_ih, lstm_0_f_b_hh)
    w0b, b0b, wh0b = dparams(lstm_0_b_w_ih, lstm_0_b_w_hh,
                             lstm_0_b_b_ih, lstm_0_b_b_hh)
    w1f, b1f, wh1f = dparams(lstm_1_f_w_ih, lstm_1_f_w_hh,
                             lstm_1_f_b_ih, lstm_1_f_b_hh)
    w1b, b1b, wh1b = dparams(lstm_1_b_w_ih, lstm_1_b_w_hh,
                             lstm_1_b_b_ih, lstm_1_b_b_hh)

    w0c = _halves(w0f[:, :, None], w0b[:, :, None], (4, 3, 64))
    b0c = _halves(b0f[:, None], b0b[:, None], (4, 64))
    whh0 = _halves(wh0f[:, None], wh0b[:, None], (4, 64)) * 0.5
    wac = _halves(w1f[:, 0:1], w1b[:, 1:2], (4, 64)) * 0.5
    wbc = _halves(w1f[:, 1:2], w1b[:, 0:1], (4, 64)) * 0.5
    b1c = _halves(b1f[:, None], b1b[:, None], (4, 64))
    whh1 = _halves(wh1f[:, None], wh1b[:, None], (4, 64)) * 0.5

    o1 = pl.pallas_call(
        functools.partial(_scan_kernel, seq=seq, unroll=UNROLL),
        out_shape=jax.ShapeDtypeStruct((seq, NC), jnp.float32),
        in_specs=[pl.BlockSpec(memory_space=pltpu.MemorySpace.VMEM)] * 8,
        out_specs=pl.BlockSpec(memory_space=pltpu.MemorySpace.VMEM),
        scratch_shapes=[pltpu.VMEM((seq, NC), jnp.float32),
                        pltpu.VMEM((UNROLL, 4, 1, NC), jnp.float32),
                        pltpu.VMEM((UNROLL, 4, 1, NC), jnp.float32)],
        compiler_params=pltpu.CompilerParams(
            vmem_limit_bytes=32 * 1024 * 1024),
    )(x2, w0c, b0c, whh0.reshape(4, 1, NC), wac, wbc, b1c,
      whh1.reshape(4, 1, NC))

    xlin = o1.reshape(seq, 2, 64).transpose(2, 0, 1).reshape(NB, 2 * seq)

    kt = 4
    tk = 2 * seq // kt
    y = pl.pallas_call(
        functools.partial(_mlp_kernel, kt=kt),
        out_shape=jax.ShapeDtypeStruct((NB, 1), jnp.float32),
        grid=(kt,),
        in_specs=[
            pl.BlockSpec((NB, tk), lambda k: (0, k)),
            pl.BlockSpec((tk, 2048), lambda k: (k, 0)),
            pl.BlockSpec((1, 2048), lambda k: (0, 0)),
            pl.BlockSpec((2048, 512), lambda k: (0, 0)),
            pl.BlockSpec((1, 512), lambda k: (0, 0)),
            pl.BlockSpec((512, 1), lambda k: (0, 0)),
            pl.BlockSpec((1, 1), lambda k: (0, 0)),
        ],
        out_specs=pl.BlockSpec((NB, 1), lambda k: (0, 0)),
        scratch_shapes=[pltpu.VMEM((NB, 2048), jnp.float32)],
        compiler_params=pltpu.CompilerParams(
            dimension_semantics=("arbitrary",),
            vmem_limit_bytes=50 * 1024 * 1024),
    )(xlin, lin1_w, lin1_b.reshape(1, -1), lin2_w,
      lin2_b.reshape(1, -1), lin3_w, lin3_b.reshape(1, -1))
    return y


# speedup vs baseline: 1.0313x; 1.0313x over previous
import functools

import jax
import jax.numpy as jnp
from jax import lax
from jax.experimental import pallas as pl
from jax.experimental.pallas import tpu as pltpu

SEQ = 4096
NB = 64
NC = 128
UNROLL = 16


def _flip_rows(x):
    u = x.shape[0]
    return jnp.concatenate([x[i:i + 1] for i in range(u - 1, -1, -1)], axis=0)


def _scan_kernel(x2_ref, w0_ref, b0_ref, whh0_ref, wa_ref, wb_ref, b1_ref,
                 whh1_ref, o1_ref, o0_scr, zb0, zb1, *, seq, unroll):
    u = unroll
    ng = seq // u
    w0 = w0_ref[...]
    b0 = b0_ref[...]
    wa = wa_ref[...]
    wb = wb_ref[...]
    b1 = b1_ref[...]
    w4_0 = tuple(whh0_ref[g] for g in range(4))
    w4_1 = tuple(whh1_ref[g] for g in range(4))


    def build0(g, buf):
        gc = jnp.minimum(g, ng - 1)
        base = gc * u
        xg = x2_ref[pl.ds(base, u), :, :]
        zg = jnp.broadcast_to(b0[None], (u, 4, NC))
        for d in range(3):
            zg = zg + xg[:, d, None, :] * w0[None, :, d, :]
        buf[...] = zg.reshape(u, 4, 1, NC)

    def build1(g, buf):
        gc = jnp.minimum(g, ng - 1)
        base = gc * u
        rbase = seq - u - base
        a_blk = o0_scr[pl.ds(base, u), :]
        r_blk = _flip_rows(o0_scr[pl.ds(rbase, u), :])
        rsh = jnp.concatenate([r_blk[:, 64:], r_blk[:, :64]], axis=-1)
        zg = (jnp.broadcast_to(b1[None], (u, 4, NC))
              + a_blk[:, None, :] * wa[None]
              + rsh[:, None, :] * wb[None])
        buf[...] = zg.reshape(u, 4, 1, NC)

    def steps(buf, st, w4, out_scale=None):
        tc, top1, c = st
        w_i, w_f, w_g, w_o = w4
        hs = []
        blocks = []
        for j in range(u):
            q_i = top1 * w_i
            q_f = top1 * w_f
            q_g = top1 * w_g
            q_o = top1 * w_o
            ch = c * 0.5
            tg = jnp.tanh(buf[j, 2] + tc * q_g)
            ti = jnp.tanh(buf[j, 0] + tc * q_i)
            tf = jnp.tanh(buf[j, 1] + tc * q_f)
            to = jnp.tanh(buf[j, 3] + tc * q_o)
            c = ch * (tf + 1.0) + (tg * 0.5) * (ti + 1.0)
            top1 = to + 1.0
            tc = jnp.tanh(c)
            h2 = tc * top1
            hs.append(h2 if out_scale is None else h2 * out_scale)
            if len(hs) == 8:
                blocks.append(jnp.concatenate(hs, axis=0))
                hs = []
        return blocks, (tc, top1, c)

    zv = jnp.zeros((1, NC), jnp.float32)
    ov = jnp.ones((1, NC), jnp.float32)

    def l0_half(g, buf_run, buf_next, st):
        build0(g + 1, buf_next)
        blocks, st = steps(buf_run, st, w4_0)
        base = g * u
        for k, blk in enumerate(blocks):
            o0_scr[pl.ds(base + 8 * k, 8), :] = blk
        return st

    def l0_body(gg, st):
        g = gg * 2
        st = l0_half(g, zb0, zb1, st)
        st = l0_half(g + 1, zb1, zb0, st)
        return st

    build0(0, zb0)
    lax.fori_loop(0, ng // 2, l0_body, (zv, ov, zv))

    def l1_half(g, buf_run, buf_next, st):
        build1(g + 1, buf_next)
        blocks, st = steps(buf_run, st, w4_1, out_scale=0.5)
        base = g * u
        rbase = seq - u - base
        for k, blk in enumerate(blocks):
            o1_ref[pl.ds(base + 8 * k, 8), 0:64] = blk[:, 0:64]
            o1_ref[pl.ds(rbase + u - 8 - 8 * k, 8), 64:128] = (
                _flip_rows(blk[:, 64:128]))
        return st

    def l1_body(gg, st):
        g = gg * 2
        st = l1_half(g, zb0, zb1, st)
        st = l1_half(g + 1, zb1, zb0, st)
        return st

    build1(0, zb0)
    lax.fori_loop(0, ng // 2, l1_body, (zv, ov, zv))


def _mlp_kernel(x_ref, w1_ref, b1_ref, w2_ref, b2_ref, w3_ref, b3_ref,
                o_ref, acc_ref, *, kt):
    k = pl.program_id(0)

    @pl.when(k == 0)
    def _():
        acc_ref[...] = jnp.zeros_like(acc_ref)

    acc_ref[...] += jnp.dot(x_ref[...], w1_ref[...],
                            preferred_element_type=jnp.float32)

    @pl.when(k == kt - 1)
    def _():
        h1 = acc_ref[...] + b1_ref[...]
        h2 = jnp.maximum(
            jnp.dot(h1, w2_ref[...], preferred_element_type=jnp.float32)
            + b2_ref[...], 0.0)
        y = jnp.dot(h2, w3_ref[...],
                    preferred_element_type=jnp.float32) + b3_ref[...]
        o_ref[...] = y


def _halves(f, b, shape):
    return jnp.concatenate([jnp.broadcast_to(f, shape),
                            jnp.broadcast_to(b, shape)], axis=-1)


def kernel(pos, batch,
           lstm_0_f_w_ih, lstm_0_f_w_hh, lstm_0_f_b_ih, lstm_0_f_b_hh,
           lstm_0_b_w_ih, lstm_0_b_w_hh, lstm_0_b_b_ih, lstm_0_b_b_hh,
           lstm_1_f_w_ih, lstm_1_f_w_hh, lstm_1_f_b_ih, lstm_1_f_b_hh,
           lstm_1_b_w_ih, lstm_1_b_w_hh, lstm_1_b_b_ih, lstm_1_b_b_hh,
           lin1_w, lin1_b, lin2_w, lin2_b, lin3_w, lin3_b):
    seq = SEQ
    x = pos.reshape(NB, seq, 3).astype(jnp.float32)
    xt = jnp.transpose(x, (1, 2, 0))
    x2 = jnp.concatenate([xt, xt[::-1]], axis=-1)

    gs = jnp.array([0.5, 0.5, 1.0, 0.5], jnp.float32)

    def dparams(w_ih, w_hh, b_ih, b_hh):
        return (w_ih * gs[:, None],
                (b_ih + b_hh) * gs,
                w_hh[:, 0] * gs)

    w0f, b0f, wh0f = dparams(lstm_0_f_w_ih, lstm_0_f_w_hh,
                             lstm_0_f_b_ih, lstm_0_f_b_hh)
    w0b, b0b, wh0b = dparams(lstm_0_b_w_ih, lstm_0_b_w_hh,
                             lstm_0_b_b_ih, lstm_0_b_b_hh)
    w1f, b1f, wh1f = dparams(lstm_1_f_w_ih, lstm_1_f_w_hh,
                             lstm_1_f_b_ih, lstm_1_f_b_hh)
    w1b, b1b, wh1b = dparams(lstm_1_b_w_ih, lstm_1_b_w_hh,
                             lstm_1_b_b_ih, lstm_1_b_b_hh)

    w0c = _halves(w0f[:, :, None], w0b[:, :, None], (4, 3, 64))
    b0c = _halves(b0f[:, None], b0b[:, None], (4, 64))
    whh0 = _halves(wh0f[:, None], wh0b[:, None], (4, 64)) * 0.5
    wac = _halves(w1f[:, 0:1], w1b[:, 1:2], (4, 64)) * 0.5
    wbc = _halves(w1f[:, 1:2], w1b[:, 0:1], (4, 64)) * 0.5
    b1c = _halves(b1f[:, None], b1b[:, None], (4, 64))
    whh1 = _halves(wh1f[:, None], wh1b[:, None], (4, 64)) * 0.5

    o1 = pl.pallas_call(
        functools.partial(_scan_kernel, seq=seq, unroll=UNROLL),
        out_shape=jax.ShapeDtypeStruct((seq, NC), jnp.float32),
        in_specs=[pl.BlockSpec(memory_space=pltpu.MemorySpace.VMEM)] * 8,
        out_specs=pl.BlockSpec(memory_space=pltpu.MemorySpace.VMEM),
        scratch_shapes=[pltpu.VMEM((seq, NC), jnp.float32),
                        pltpu.VMEM((UNROLL, 4, 1, NC), jnp.float32),
                        pltpu.VMEM((UNROLL, 4, 1, NC), jnp.float32)],
        compiler_params=pltpu.CompilerParams(
            vmem_limit_bytes=32 * 1024 * 1024),
    )(x2, w0c, b0c, whh0.reshape(4, 1, NC), wac, wbc, b1c,
      whh1.reshape(4, 1, NC))

    xlin = o1.reshape(seq, 2, 64).transpose(2, 0, 1).reshape(NB, 2 * seq)

    kt = 4
    tk = 2 * seq // kt
    y = pl.pallas_call(
        functools.partial(_mlp_kernel, kt=kt),
        out_shape=jax.ShapeDtypeStruct((NB, 1), jnp.float32),
        grid=(kt,),
        in_specs=[
            pl.BlockSpec((NB, tk), lambda k: (0, k)),
            pl.BlockSpec((tk, 2048), lambda k: (k, 0)),
            pl.BlockSpec((1, 2048), lambda k: (0, 0)),
            pl.BlockSpec((2048, 512), lambda k: (0, 0)),
            pl.BlockSpec((1, 512), lambda k: (0, 0)),
            pl.BlockSpec((512, 1), lambda k: (0, 0)),
            pl.BlockSpec((1, 1), lambda k: (0, 0)),
        ],
        out_specs=pl.BlockSpec((NB, 1), lambda k: (0, 0)),
        scratch_shapes=[pltpu.VMEM((NB, 2048), jnp.float32)],
        compiler_params=pltpu.CompilerParams(
            dimension_semantics=("arbitrary",),
            vmem_limit_bytes=50 * 1024 * 1024),
    )(xlin, lin1_w, lin1_b.reshape(1, -1), lin2_w,
      lin2_b.reshape(1, -1), lin3_w, lin3_b.reshape(1, -1))
    return y
